# baseline (speedup 1.0000x reference)
import jax
import jax.numpy as jnp
from jax import lax
from jax.experimental import pallas as pl
from jax.experimental.pallas import tpu as pltpu
from jax.experimental.pallas import tpu_sc as plsc

_L = 16
_W = 128
_B = 512
_PER = _B // _L
_NREG = _PER // _L
_SOFF = 4 * _L


def _gan_loss_body(probs_hbm, tgt_hbm, rew_hbm, out_hbm,
                   tgt_vm, rew_vm, val_vm, acc_vm, pacc_vm, tot_vm,
                   shared, sem):
    s = lax.axis_index("s")
    base = s * _PER
    cp_t = pltpu.async_copy(tgt_hbm.at[pl.ds(base, _PER)], tgt_vm, sem)
    cp_r = pltpu.async_copy(rew_hbm.at[pl.ds(base, _PER)], rew_vm, sem)
    cp_t.wait()
    tvecs = [tgt_vm[pl.ds(j * _L, _L)] for j in range(_NREG)]
    copies = []
    for k in range(_PER):
        t = tvecs[k // _L][k % _L]
        c0 = pl.multiple_of(
            lax.shift_left(lax.shift_right_logical(t, 7), 7), _W)
        r0 = pl.multiple_of(base + (k & ~7), 8)
        copies.append(pltpu.async_copy(
            probs_hbm.at[pl.ds(r0, 8), pl.ds(c0, _W)],
            val_vm.at[pl.ds(k * 8, 8), :], sem))
    cp_r.wait()
    for cp in copies:
        cp.wait()
    acc = jnp.zeros((_L,), jnp.float32)
    sub = lax.bitwise_and(lax.iota(jnp.int32, _L), 7)
    for j in range(_NREG):
        rid = (j * _L + lax.iota(jnp.int32, _L)) * 8 + sub
        cid = lax.bitwise_and(tvecs[j], _W - 1)
        vals = plsc.load_gather(val_vm, [rid, cid])
        acc = acc - vals * rew_vm[pl.ds(j * _L, _L)]
    acc_vm[...] = acc
    pltpu.sync_copy(acc_vm, shared.at[_SOFF + s])
    plsc.subcore_barrier()

    @pl.when(s == 0)
    def _reduce():
        pltpu.sync_copy(shared.at[pl.ds(_SOFF, _L)], pacc_vm)
        tot = jnp.zeros((_L,), jnp.float32)
        for i in range(_L):
            tot = tot + pacc_vm[i]
        total = jnp.sum(tot)
        tot_vm[...] = lax.broadcast_in_dim(total, (_L,), ())
        pltpu.sync_copy(tot_vm, out_hbm)


@jax.jit
def _gan_loss(probs2d, targets, rewards):
    mesh = plsc.VectorSubcoreMesh(core_axis_name="c", subcore_axis_name="s",
                                  num_cores=1)
    launcher = pl.kernel(
        _gan_loss_body,
        mesh=mesh,
        out_type=jax.ShapeDtypeStruct((_L,), jnp.float32),
        compiler_params=pltpu.CompilerParams(needs_layout_passes=False),
        scratch_types=[
            pltpu.VMEM((_PER,), jnp.int32),
            pltpu.VMEM((_PER,), jnp.float32),
            pltpu.VMEM((_PER * 8, _W), jnp.float32),
            pltpu.VMEM((_L,), jnp.float32),
            pltpu.VMEM((_L, _L), jnp.float32),
            pltpu.VMEM((_L,), jnp.float32),
            pltpu.VMEM_SHARED((_SOFF + _L, _L), jnp.float32),
            pltpu.SemaphoreType.DMA,
        ],
    )
    return launcher(probs2d, targets, rewards)


def kernel(probs, targets, rewards):
    vocab = probs.shape[-1]
    probs2d = probs.reshape(-1, vocab)
    out = _gan_loss(probs2d, targets, rewards)
    return out[0]
